# baseline (speedup 1.0000x reference)
import jax
import jax.numpy as jnp
from jax.experimental import pallas as pl
from jax.experimental.pallas import tpu as pltpu

_NUM_EXPERTS = 64
_TOP_K = 2
_HIDDEN = 4096
_BT = 512
_NBUF = 4
_OBUF = 4


def _router_body(x_hbm, w_ref, tp_hbm, ti_hbm, lg_hbm,
                 x_buf, tp_buf, ti_buf, lg_buf,
                 in_sems, tp_sems, ti_sems, lg_sems):
    n_tokens = x_hbm.shape[0]
    nchunks = n_tokens // _BT
    w = w_ref[...]

    def in_copy(c, slot):
        return pltpu.make_async_copy(
            x_hbm.at[pl.ds(c * _BT, _BT), :], x_buf.at[slot], in_sems.at[slot])

    def out_copies(c, slot):
        rows = pl.ds(c * _BT, _BT)
        return (
            pltpu.make_async_copy(tp_buf.at[slot], tp_hbm.at[rows, :], tp_sems.at[slot]),
            pltpu.make_async_copy(ti_buf.at[slot], ti_hbm.at[rows, :], ti_sems.at[slot]),
            pltpu.make_async_copy(lg_buf.at[slot], lg_hbm.at[rows, :], lg_sems.at[slot]),
        )

    for s in range(min(_NBUF, nchunks)):
        in_copy(s, s).start()

    for c in range(nchunks):
        slot = c % _NBUF
        in_copy(c, slot).wait()
        x = x_buf[slot]
        logits = jax.lax.dot_general(
            x, w,
            dimension_numbers=(((1,), (1,)), ((), ())),
            preferred_element_type=jnp.float32,
            precision=jax.lax.Precision.DEFAULT,
        )

        m1 = jnp.max(logits, axis=1, keepdims=True)
        e = jnp.exp(logits - m1)
        z = jnp.sum(e, axis=1, keepdims=True)

        iota = jax.lax.broadcasted_iota(jnp.int32, logits.shape, 1)
        sentinel = jnp.int32(_NUM_EXPERTS)
        i1 = jnp.min(jnp.where(logits == m1, iota, sentinel), axis=1, keepdims=True)
        masked = jnp.where(iota == i1, -jnp.inf, logits)
        m2 = jnp.max(masked, axis=1, keepdims=True)
        i2 = jnp.min(jnp.where(masked == m2, iota, sentinel), axis=1, keepdims=True)

        p1 = jnp.exp(m1 - m1) / z
        p2 = jnp.exp(m2 - m1) / z

        oslot = c % _OBUF
        if c >= _OBUF:
            for cp in out_copies(c - _OBUF, oslot):
                cp.wait()
        lg_buf[oslot] = logits
        tp_buf[oslot] = jnp.concatenate([p1, p2], axis=1)
        ti_buf[oslot] = jnp.concatenate([i1, i2], axis=1)
        for cp in out_copies(c, oslot):
            cp.start()

        nxt = c + _NBUF
        if nxt < nchunks:
            in_copy(nxt, slot).start()

    for c in range(max(0, nchunks - _OBUF), nchunks):
        for cp in out_copies(c, c % _OBUF):
            cp.wait()


@jax.jit
def kernel(input, weight):
    n_tokens = input.shape[0]
    tp, ti, lg = pl.pallas_call(
        _router_body,
        in_specs=[
            pl.BlockSpec(memory_space=pl.ANY),
            pl.BlockSpec((_NUM_EXPERTS, _HIDDEN), lambda: (0, 0)),
        ],
        out_specs=[
            pl.BlockSpec(memory_space=pl.ANY),
            pl.BlockSpec(memory_space=pl.ANY),
            pl.BlockSpec(memory_space=pl.ANY),
        ],
        out_shape=[
            jax.ShapeDtypeStruct((n_tokens, _TOP_K), jnp.float32),
            jax.ShapeDtypeStruct((n_tokens, _TOP_K), jnp.int32),
            jax.ShapeDtypeStruct((n_tokens, _NUM_EXPERTS), jnp.float32),
        ],
        scratch_shapes=[
            pltpu.VMEM((_NBUF, _BT, _HIDDEN), jnp.float32),
            pltpu.VMEM((_OBUF, _BT, _TOP_K), jnp.float32),
            pltpu.VMEM((_OBUF, _BT, _TOP_K), jnp.int32),
            pltpu.VMEM((_OBUF, _BT, _NUM_EXPERTS), jnp.float32),
            pltpu.SemaphoreType.DMA((_NBUF,)),
            pltpu.SemaphoreType.DMA((_OBUF,)),
            pltpu.SemaphoreType.DMA((_OBUF,)),
            pltpu.SemaphoreType.DMA((_OBUF,)),
        ],
        compiler_params=pltpu.CompilerParams(
            vmem_limit_bytes=60 * 1024 * 1024,
        ),
    )(input, weight)
    return tp, ti, lg
